# Initial kernel scaffold; baseline (speedup 1.0000x reference)
import jax
import jax.numpy as jnp
from jax.experimental import pallas as pl


def kernel(x):
    raise NotImplementedError("write your pallas kernel here")


# speedup vs baseline: 13.7744x; 13.7744x over previous
import functools

import jax
import jax.numpy as jnp
from jax import lax
from jax.experimental import pallas as pl

K = 10
RB = 256


def _knn_hist_block(xb_ref, xa_ref, out_ref):
    xb = xb_ref[...]
    xa = xa_ref[...]
    n = xa.shape[0]
    s = out_ref.shape[-1]

    sqb = jnp.sum(xb * xb, axis=1, keepdims=True)
    sqa = jnp.sum(xa * xa, axis=1, keepdims=True)
    sqa_row = sqa.reshape(1, n)
    dot = lax.dot_general(xb, xa, (((1,), (1,)), ((), ())),
                          preferred_element_type=jnp.float32)
    d2 = sqb + sqa_row - 2.0 * dot
    d2 = jnp.maximum(d2, 0.0)

    iota_c = lax.broadcasted_iota(jnp.int32, (RB, n), 1)
    iota_s = lax.broadcasted_iota(jnp.int32, (RB, s), 1)
    counts = jnp.zeros((RB, s), jnp.float32)
    d = d2
    for _ in range(K):
        m = jnp.min(d, axis=1, keepdims=True)
        idx = jnp.min(jnp.where(d == m, iota_c, n), axis=1,
                      keepdims=True)
        counts += (jnp.bitwise_and(idx, s - 1) == iota_s).astype(jnp.float32)
        d = jnp.where(iota_c == idx, jnp.float32(jnp.inf), d)

    out_ref[...] = counts.reshape(RB // s, s, s)


@jax.jit
def kernel(x):
    b, s, d_dim = x.shape
    n = b * s
    flat = x.reshape(n, d_dim)
    grid = n // RB
    out = pl.pallas_call(
        _knn_hist_block,
        grid=(grid,),
        in_specs=[
            pl.BlockSpec((RB, d_dim), lambda g: (g, 0)),
            pl.BlockSpec((n, d_dim), lambda g: (0, 0)),
        ],
        out_specs=pl.BlockSpec((RB // s, s, s), lambda g: (g, 0, 0)),
        out_shape=jax.ShapeDtypeStruct((b, s, s), jnp.float32),
    )(flat, flat)
    return out


# speedup vs baseline: 14.5255x; 1.0545x over previous
import jax
import jax.numpy as jnp
from jax import lax
from jax.experimental import pallas as pl

K = 10
RB = 256


def _knn_hist_block(xb_ref, xa_ref, out_ref):
    xb = xb_ref[...]
    xa = xa_ref[...]
    n = xa.shape[0]
    s = out_ref.shape[-1]
    inf = jnp.float32(jnp.inf)

    sqb = jnp.sum(xb * xb, axis=1, keepdims=True)
    sqa = jnp.sum(xa * xa, axis=1, keepdims=True)
    sqa_row = sqa.reshape(1, n)
    dot = lax.dot_general(xb, xa, (((1,), (1,)), ((), ())),
                          preferred_element_type=jnp.float32)
    d = jnp.maximum(sqb + sqa_row - 2.0 * dot, 0.0)

    levels = []
    lo = jnp.full((RB, 1), -1.0, jnp.float32)
    for _ in range(K):
        lo = jnp.min(jnp.where(d > lo, d, inf), axis=1, keepdims=True)
        levels.append(lo)

    c_le = jnp.sum((d <= levels[K - 2]).astype(jnp.float32), axis=1,
                   keepdims=True)
    v = jnp.where(c_le >= jnp.float32(K), levels[K - 2], levels[K - 1])

    d3 = d.reshape(RB, n // s, s)
    hist_lt = jnp.sum((d3 < v.reshape(RB, 1, 1)).astype(jnp.float32),
                      axis=1)
    n_lt = jnp.sum(hist_lt, axis=1, keepdims=True)

    iota_c = lax.broadcasted_iota(jnp.int32, (RB, n), 1)
    iota_s = lax.broadcasted_iota(jnp.int32, (RB, s), 1)
    e = jnp.where(d == v, iota_c, n)
    i1 = jnp.min(e, axis=1, keepdims=True)
    i2 = jnp.min(jnp.where(e == i1, n, e), axis=1, keepdims=True)
    r = jnp.float32(K) - n_lt
    pick1 = ((jnp.bitwise_and(i1, s - 1) == iota_s)
             & (r >= 1.0)).astype(jnp.float32)
    pick2 = ((jnp.bitwise_and(i2, s - 1) == iota_s)
             & (r >= 2.0)).astype(jnp.float32)

    out_ref[...] = (hist_lt + pick1 + pick2).reshape(RB // s, s, s)


@jax.jit
def kernel(x):
    b, s, d_dim = x.shape
    n = b * s
    flat = x.reshape(n, d_dim)
    grid = n // RB
    out = pl.pallas_call(
        _knn_hist_block,
        grid=(grid,),
        in_specs=[
            pl.BlockSpec((RB, d_dim), lambda g: (g, 0)),
            pl.BlockSpec((n, d_dim), lambda g: (0, 0)),
        ],
        out_specs=pl.BlockSpec((RB // s, s, s), lambda g: (g, 0, 0)),
        out_shape=jax.ShapeDtypeStruct((b, s, s), jnp.float32),
    )(flat, flat)
    return out


# speedup vs baseline: 19.3904x; 1.3349x over previous
import jax
import jax.numpy as jnp
from jax import lax
from jax.experimental import pallas as pl

K = 10
RB = 256


def _knn_hist_block(xb_ref, xa_ref, out_ref):
    xb = xb_ref[...]
    xa = xa_ref[...]
    n = xa.shape[0]
    s = out_ref.shape[-1]
    inf = jnp.float32(jnp.inf)

    sqb = jnp.sum(xb * xb, axis=1, keepdims=True)
    sqa = jnp.sum(xa * xa, axis=1, keepdims=True)
    sqa_row = sqa.reshape(1, n)
    dot = lax.dot_general(xb, xa, (((1,), (1,)), ((), ())),
                          preferred_element_type=jnp.float32)
    d = jnp.maximum(sqb + sqa_row - 2.0 * dot, 0.0)

    levels = []
    lo = jnp.full((RB, 1), -1.0, jnp.float32)
    for _ in range(K):
        lo = jnp.min(jnp.where(d > lo, d, inf), axis=1, keepdims=True)
        levels.append(lo)

    a_mat = (jnp.bitwise_and(lax.broadcasted_iota(jnp.int32, (n, s), 0), s - 1)
             == lax.broadcasted_iota(jnp.int32, (n, s), 1)
             ).astype(jnp.bfloat16)

    le9 = (d <= levels[K - 2]).astype(jnp.bfloat16)
    c_le = jnp.sum(lax.dot_general(le9, a_mat, (((1,), (0,)), ((), ())),
                                   preferred_element_type=jnp.float32),
                   axis=1, keepdims=True)
    v = jnp.where(c_le >= jnp.float32(K), levels[K - 2], levels[K - 1])

    ltv = (d < v).astype(jnp.bfloat16)
    hist_lt = lax.dot_general(ltv, a_mat, (((1,), (0,)), ((), ())),
                              preferred_element_type=jnp.float32)
    n_lt = jnp.sum(hist_lt, axis=1, keepdims=True)

    iota_c = lax.broadcasted_iota(jnp.int32, (RB, n), 1)
    iota_s = lax.broadcasted_iota(jnp.int32, (RB, s), 1)
    e = jnp.where(d == v, iota_c, n)
    i1 = jnp.min(e, axis=1, keepdims=True)
    i2 = jnp.min(jnp.where(e == i1, n, e), axis=1, keepdims=True)
    r = jnp.float32(K) - n_lt
    pick1 = ((jnp.bitwise_and(i1, s - 1) == iota_s)
             & (r >= 1.0)).astype(jnp.float32)
    pick2 = ((jnp.bitwise_and(i2, s - 1) == iota_s)
             & (r >= 2.0)).astype(jnp.float32)

    out_ref[...] = (hist_lt + pick1 + pick2).reshape(RB // s, s, s)


@jax.jit
def kernel(x):
    b, s, d_dim = x.shape
    n = b * s
    flat = x.reshape(n, d_dim)
    grid = n // RB
    out = pl.pallas_call(
        _knn_hist_block,
        grid=(grid,),
        in_specs=[
            pl.BlockSpec((RB, d_dim), lambda g: (g, 0)),
            pl.BlockSpec((n, d_dim), lambda g: (0, 0)),
        ],
        out_specs=pl.BlockSpec((RB // s, s, s), lambda g: (g, 0, 0)),
        out_shape=jax.ShapeDtypeStruct((b, s, s), jnp.float32),
    )(flat, flat)
    return out


# speedup vs baseline: 23.1507x; 1.1939x over previous
import jax
import jax.numpy as jnp
from jax import lax
from jax.experimental import pallas as pl

K = 10
RB = 256


def _knn_hist_block(xb_ref, xa_ref, out_ref):
    xb = xb_ref[...]
    xa = xa_ref[...]
    n = xa.shape[0]
    s = out_ref.shape[-1]
    inf = jnp.float32(jnp.inf)

    sqb = jnp.sum(xb * xb, axis=1, keepdims=True)
    sqa = jnp.sum(xa * xa, axis=1, keepdims=True)
    sqa_row = sqa.reshape(1, n)
    dotn = lax.dot_general(-2.0 * xb, xa, (((1,), (1,)), ((), ())),
                           preferred_element_type=jnp.float32)
    d = jnp.maximum(sqb + sqa_row + dotn, 0.0)

    levels = []
    lo = jnp.full((RB, 1), -1.0, jnp.float32)
    for _ in range(K):
        lo = jnp.min(jnp.where(d > lo, d, inf), axis=1, keepdims=True)
        levels.append(lo)

    a_mat = (jnp.bitwise_and(lax.broadcasted_iota(jnp.int32, (n, s), 0), s - 1)
             == lax.broadcasted_iota(jnp.int32, (n, s), 1)
             ).astype(jnp.bfloat16)

    le9 = (d <= levels[K - 2]).astype(jnp.bfloat16)
    c_le = jnp.sum(lax.dot_general(le9, a_mat, (((1,), (0,)), ((), ())),
                                   preferred_element_type=jnp.float32),
                   axis=1, keepdims=True)
    v = jnp.where(c_le >= jnp.float32(K), levels[K - 2], levels[K - 1])

    lev = (d <= v).astype(jnp.bfloat16)
    hist = lax.dot_general(lev, a_mat, (((1,), (0,)), ((), ())),
                           preferred_element_type=jnp.float32)

    out_ref[...] = hist.reshape(RB // s, s, s)


@jax.jit
def kernel(x):
    b, s, d_dim = x.shape
    n = b * s
    flat = x.reshape(n, d_dim)
    grid = n // RB
    out = pl.pallas_call(
        _knn_hist_block,
        grid=(grid,),
        in_specs=[
            pl.BlockSpec((RB, d_dim), lambda g: (g, 0)),
            pl.BlockSpec((n, d_dim), lambda g: (0, 0)),
        ],
        out_specs=pl.BlockSpec((RB // s, s, s), lambda g: (g, 0, 0)),
        out_shape=jax.ShapeDtypeStruct((b, s, s), jnp.float32),
    )(flat, flat)
    return out


# speedup vs baseline: 23.3330x; 1.0079x over previous
import jax
import jax.numpy as jnp
from jax import lax
from jax.experimental import pallas as pl

K = 10
RB = 512


def _knn_hist_block(xb_ref, xa_ref, out_ref):
    xb = xb_ref[...]
    xa = xa_ref[...]
    n = xa.shape[0]
    s = out_ref.shape[-1]
    inf = jnp.float32(jnp.inf)

    sqb = jnp.sum(xb * xb, axis=1, keepdims=True)
    sqa = jnp.sum(xa * xa, axis=1, keepdims=True)
    sqa_row = sqa.reshape(1, n)
    dotn = lax.dot_general(-2.0 * xb, xa, (((1,), (1,)), ((), ())),
                           preferred_element_type=jnp.float32)
    d = jnp.maximum(sqb + sqa_row + dotn, 0.0)

    levels = []
    lo = jnp.full((RB, 1), -1.0, jnp.float32)
    for _ in range(K):
        lo = jnp.min(jnp.where(d > lo, d, inf), axis=1, keepdims=True)
        levels.append(lo)

    a_mat = (jnp.bitwise_and(lax.broadcasted_iota(jnp.int32, (n, s), 0), s - 1)
             == lax.broadcasted_iota(jnp.int32, (n, s), 1)
             ).astype(jnp.bfloat16)

    le9 = (d <= levels[K - 2]).astype(jnp.bfloat16)
    c_le = jnp.sum(lax.dot_general(le9, a_mat, (((1,), (0,)), ((), ())),
                                   preferred_element_type=jnp.float32),
                   axis=1, keepdims=True)
    v = jnp.where(c_le >= jnp.float32(K), levels[K - 2], levels[K - 1])

    lev = (d <= v).astype(jnp.bfloat16)
    hist = lax.dot_general(lev, a_mat, (((1,), (0,)), ((), ())),
                           preferred_element_type=jnp.float32)

    out_ref[...] = hist.reshape(RB // s, s, s)


@jax.jit
def kernel(x):
    b, s, d_dim = x.shape
    n = b * s
    flat = x.reshape(n, d_dim)
    grid = n // RB
    out = pl.pallas_call(
        _knn_hist_block,
        grid=(grid,),
        in_specs=[
            pl.BlockSpec((RB, d_dim), lambda g: (g, 0)),
            pl.BlockSpec((n, d_dim), lambda g: (0, 0)),
        ],
        out_specs=pl.BlockSpec((RB // s, s, s), lambda g: (g, 0, 0)),
        out_shape=jax.ShapeDtypeStruct((b, s, s), jnp.float32),
    )(flat, flat)
    return out
